# Initial kernel scaffold; baseline (speedup 1.0000x reference)
import jax
import jax.numpy as jnp
from jax.experimental import pallas as pl


def kernel(hkl, miller_id, seen):
    raise NotImplementedError("write your pallas kernel here")


# speedup vs baseline: 14.8314x; 14.8314x over previous
import functools

import jax
import jax.numpy as jnp
from jax import lax
from jax.experimental import pallas as pl
from jax.experimental.pallas import tpu as pltpu
from jax.experimental.pallas import tpu_sc as plsc

N = 2_000_000
G = 161
GRID_FLAT = G * G * G
ASU = 1_000_000

NC = 2
NS = 16
NW = NC * NS

PADN = 1 << 21
PW = PADN // NW
CH = 2048
SEEN_PAD = 1 << 20
PER_TILE_SEEN = SEEN_PAD // NS
ZB = 8192

_PAD_F = G * G * G


def _linearize_body(h_ref, k_ref, l_ref, f_ref):
    f_ref[...] = h_ref[...] * (G * G) + k_ref[...] * G + l_ref[...]


def _merge_body(a_ref, b_ref, s_ref, o_ref):
    o_ref[...] = (a_ref[...] | b_ref[...] | s_ref[...]) != 0


def _sc_body(f_hbm, miller_hbm, out_hbm, seen2_hbm,
             idx_v, vals_v, ones_v, zer_v, seen_sp):
    c = lax.axis_index("c")
    s = lax.axis_index("s")
    wid = c * NS + s

    def fill_ones(i, carry):
        ones_v[pl.ds(i * 16, 16)] = jnp.full((16,), 1, jnp.int32)
        return carry

    lax.fori_loop(0, CH // 16, fill_ones, 0)

    def fill_zeros(i, carry):
        zer_v[pl.ds(i * 16, 16)] = jnp.zeros((16,), jnp.int32)
        return carry

    lax.fori_loop(0, ZB // 16, fill_zeros, 0)

    def zero_seen(i, carry):
        pltpu.sync_copy(zer_v,
                        seen_sp.at[pl.ds(s * PER_TILE_SEEN + i * ZB, ZB)])
        return carry

    lax.fori_loop(0, PER_TILE_SEEN // ZB, zero_seen, 0)
    plsc.subcore_barrier()

    def chunk(j, carry):
        base = wid * PW + j * CH
        pltpu.sync_copy(f_hbm.at[pl.ds(base, CH)], idx_v)
        pltpu.sync_copy(miller_hbm.at[idx_v], vals_v)
        pltpu.sync_copy(vals_v, out_hbm.at[pl.ds(base, CH)])
        pltpu.sync_copy(ones_v, seen_sp.at[vals_v])
        return carry

    lax.fori_loop(0, PW // CH, chunk, 0)
    plsc.subcore_barrier()

    pltpu.sync_copy(seen_sp.at[pl.ds(s * PER_TILE_SEEN, PER_TILE_SEEN)],
                    seen2_hbm.at[c, pl.ds(s * PER_TILE_SEEN, PER_TILE_SEEN)])


@functools.partial(
    pl.kernel,
    out_type=(
        jax.ShapeDtypeStruct((PADN,), jnp.int32),
        jax.ShapeDtypeStruct((NC, SEEN_PAD), jnp.int32),
    ),
    mesh=plsc.VectorSubcoreMesh(core_axis_name="c", subcore_axis_name="s"),
    scratch_types=[
        pltpu.VMEM((CH,), jnp.int32),
        pltpu.VMEM((CH,), jnp.int32),
        pltpu.VMEM((CH,), jnp.int32),
        pltpu.VMEM((ZB,), jnp.int32),
        pltpu.VMEM_SHARED((SEEN_PAD,), jnp.int32),
    ],
)
def _sc_gather_scatter(f_hbm, miller_hbm, out_hbm, seen2_hbm,
                       idx_v, vals_v, ones_v, zer_v, seen_sp):
    _sc_body(f_hbm, miller_hbm, out_hbm, seen2_hbm,
             idx_v, vals_v, ones_v, zer_v, seen_sp)


def kernel(hkl, miller_id, seen):
    hkl = hkl.astype(jnp.int32)
    pad = PADN - N
    h = jnp.concatenate([hkl[:, 0], jnp.full((pad,), G, jnp.int32)])
    k = jnp.concatenate([hkl[:, 1], jnp.zeros((pad,), jnp.int32)])
    l = jnp.concatenate([hkl[:, 2], jnp.zeros((pad,), jnp.int32)])
    shape2d = (PADN // 1024, 1024)
    h2, k2, l2 = h.reshape(shape2d), k.reshape(shape2d), l.reshape(shape2d)

    f2 = pl.pallas_call(
        _linearize_body,
        out_shape=jax.ShapeDtypeStruct(shape2d, jnp.int32),
        grid=(16,),
        in_specs=[pl.BlockSpec((shape2d[0] // 16, 1024), lambda i: (i, 0))] * 3,
        out_specs=pl.BlockSpec((shape2d[0] // 16, 1024), lambda i: (i, 0)),
    )(h2, k2, l2)
    f = f2.reshape(PADN)

    miller_p = jnp.concatenate(
        [miller_id.reshape(-1), jnp.full((7,), ASU, jnp.int32)])

    out_p, seen2 = _sc_gather_scatter(f, miller_p)

    seen32 = jnp.concatenate(
        [seen.astype(jnp.int32), jnp.zeros((SEEN_PAD - ASU,), jnp.int32)])
    mshape = (SEEN_PAD // 1024, 1024)
    merged = pl.pallas_call(
        _merge_body,
        out_shape=jax.ShapeDtypeStruct(mshape, jnp.bool_),
        grid=(8,),
        in_specs=[pl.BlockSpec((mshape[0] // 8, 1024), lambda i: (i, 0))] * 3,
        out_specs=pl.BlockSpec((mshape[0] // 8, 1024), lambda i: (i, 0)),
    )(seen2[0].reshape(mshape), seen2[1].reshape(mshape),
      seen32.reshape(mshape))

    return out_p[:N], merged.reshape(-1)[:ASU]
